# baseline (speedup 1.0000x reference)
import jax
import jax.numpy as jnp
from jax.experimental import pallas as pl
from jax.experimental.pallas import tpu as pltpu

N = 10000
D_IN = 128
D_OUT = 32

ROW_BLOCK = 512


def _fused_kernel(x_ref, w_ref, adj_ref, o_ref, xw_ref):
    @pl.when(pl.program_id(0) == 0)
    def _():
        xw_ref[...] = jax.lax.dot_general(
            x_ref[...], w_ref[...],
            dimension_numbers=(((1,), (0,)), ((), ())),
            preferred_element_type=jnp.float32,
        )

    acc = jax.lax.dot_general(
        adj_ref[...], xw_ref[...],
        dimension_numbers=(((1,), (0,)), ((), ())),
        preferred_element_type=jnp.float32,
    )
    o_ref[...] = jnp.maximum(acc, 0.0)


def kernel(inputs, adj_norm, weight):
    grid = ((N + ROW_BLOCK - 1) // ROW_BLOCK,)
    out = pl.pallas_call(
        _fused_kernel,
        grid=grid,
        in_specs=[
            pl.BlockSpec((N, D_IN), lambda i: (0, 0)),
            pl.BlockSpec((D_IN, D_OUT), lambda i: (0, 0)),
            pl.BlockSpec((ROW_BLOCK, N), lambda i: (i, 0)),
        ],
        out_specs=pl.BlockSpec((ROW_BLOCK, D_OUT), lambda i: (i, 0)),
        out_shape=jax.ShapeDtypeStruct((N, D_OUT), jnp.float32),
        scratch_shapes=[pltpu.VMEM((N, D_OUT), jnp.float32)],
        compiler_params=pltpu.CompilerParams(
            dimension_semantics=("arbitrary",),
        ),
    )(inputs, weight, adj_norm)
    return out


# speedup vs baseline: 1.0797x; 1.0797x over previous
import jax
import jax.numpy as jnp
from jax.experimental import pallas as pl
from jax.experimental.pallas import tpu as pltpu

N = 10000
D_IN = 128
D_OUT = 32

ROW_BLOCK = 400


def _fused_kernel(x_ref, wt_ref, adj_ref, ot_ref, xw_ref, acc_ref):
    i = pl.program_id(0)

    @pl.when(i == 0)
    def _():
        xw_ref[...] = jax.lax.dot_general(
            x_ref[...], wt_ref[...],
            dimension_numbers=(((1,), (1,)), ((), ())),
            preferred_element_type=jnp.float32,
        )

    acc = jax.lax.dot_general(
        adj_ref[...], xw_ref[...],
        dimension_numbers=(((1,), (0,)), ((), ())),
        preferred_element_type=jnp.float32,
    )
    acc_ref[pl.ds(i * ROW_BLOCK, ROW_BLOCK), :] = jnp.maximum(acc, 0.0)

    @pl.when(i == pl.num_programs(0) - 1)
    def _():
        ot_ref[...] = acc_ref[...].T


def kernel(inputs, adj_norm, weight):
    grid = (N // ROW_BLOCK,)
    out_t = pl.pallas_call(
        _fused_kernel,
        grid=grid,
        in_specs=[
            pl.BlockSpec((N, D_IN), lambda i: (0, 0)),
            pl.BlockSpec((D_OUT, D_IN), lambda i: (0, 0)),
            pl.BlockSpec((ROW_BLOCK, N), lambda i: (i, 0)),
        ],
        out_specs=pl.BlockSpec((D_OUT, N), lambda i: (0, 0)),
        out_shape=jax.ShapeDtypeStruct((D_OUT, N), jnp.float32),
        scratch_shapes=[
            pltpu.VMEM((N, D_OUT), jnp.float32),
            pltpu.VMEM((N, D_OUT), jnp.float32),
        ],
        compiler_params=pltpu.CompilerParams(
            dimension_semantics=("arbitrary",),
        ),
    )(inputs, weight.T, adj_norm)
    return out_t.T


# speedup vs baseline: 1.0812x; 1.0014x over previous
import jax
import jax.numpy as jnp
from jax.experimental import pallas as pl
from jax.experimental.pallas import tpu as pltpu

N = 10000
D_IN = 128
D_OUT = 32

ROW_BLOCK = 400


def _fused_kernel(x_ref, wt_ref, adj_ref, ot_ref, xw_ref, acc_ref):
    i = pl.program_id(0)

    @pl.when(i == 0)
    def _():
        xw_ref[...] = jax.lax.dot_general(
            x_ref[...], wt_ref[...],
            dimension_numbers=(((1,), (1,)), ((), ())),
            preferred_element_type=jnp.float32,
        ).astype(jnp.bfloat16)

    acc = jax.lax.dot_general(
        adj_ref[...], xw_ref[...],
        dimension_numbers=(((1,), (0,)), ((), ())),
        preferred_element_type=jnp.float32,
    )
    acc_ref[pl.ds(i * ROW_BLOCK, ROW_BLOCK), :] = jnp.maximum(acc, 0.0)

    @pl.when(i == pl.num_programs(0) - 1)
    def _():
        ot_ref[...] = acc_ref[...].T


def kernel(inputs, adj_norm, weight):
    grid = (N // ROW_BLOCK,)
    out_t = pl.pallas_call(
        _fused_kernel,
        grid=grid,
        in_specs=[
            pl.BlockSpec((N, D_IN), lambda i: (0, 0)),
            pl.BlockSpec((D_OUT, D_IN), lambda i: (0, 0)),
            pl.BlockSpec((ROW_BLOCK, N), lambda i: (i, 0)),
        ],
        out_specs=pl.BlockSpec((D_OUT, N), lambda i: (0, 0)),
        out_shape=jax.ShapeDtypeStruct((D_OUT, N), jnp.float32),
        scratch_shapes=[
            pltpu.VMEM((N, D_OUT), jnp.bfloat16),
            pltpu.VMEM((N, D_OUT), jnp.float32),
        ],
        compiler_params=pltpu.CompilerParams(
            dimension_semantics=("arbitrary",),
        ),
    )(inputs, weight.T, adj_norm)
    return out_t.T


# speedup vs baseline: 1.0917x; 1.0098x over previous
import jax
import jax.numpy as jnp
from jax.experimental import pallas as pl
from jax.experimental.pallas import tpu as pltpu

N = 10000
D_IN = 128
D_OUT = 32

ROW_BLOCK = 400
NB = N // ROW_BLOCK


def _fused_kernel(x_ref, wt_ref, adj_ref, ot_ref, xw_ref, tile_ref):
    i = pl.program_id(0)

    @pl.when(i == 0)
    def _():
        xw_ref[...] = jax.lax.dot_general(
            x_ref[...], wt_ref[...],
            dimension_numbers=(((1,), (1,)), ((), ())),
            preferred_element_type=jnp.float32,
        ).astype(jnp.bfloat16)

    def store_t(c):
        ot_ref[:, c * ROW_BLOCK:(c + 1) * ROW_BLOCK] = tile_ref[...].T

    for c in range(NB - 1):
        pl.when(i == c + 1)(lambda c=c: store_t(c))

    acc = jax.lax.dot_general(
        adj_ref[...], xw_ref[...],
        dimension_numbers=(((1,), (0,)), ((), ())),
        preferred_element_type=jnp.float32,
    )
    tile_ref[...] = jnp.maximum(acc, 0.0)

    @pl.when(i == NB - 1)
    def _():
        store_t(NB - 1)


def kernel(inputs, adj_norm, weight):
    grid = (NB,)
    out_t = pl.pallas_call(
        _fused_kernel,
        grid=grid,
        in_specs=[
            pl.BlockSpec((N, D_IN), lambda i: (0, 0)),
            pl.BlockSpec((D_OUT, D_IN), lambda i: (0, 0)),
            pl.BlockSpec((ROW_BLOCK, N), lambda i: (i, 0)),
        ],
        out_specs=pl.BlockSpec((D_OUT, N), lambda i: (0, 0)),
        out_shape=jax.ShapeDtypeStruct((D_OUT, N), jnp.float32),
        scratch_shapes=[
            pltpu.VMEM((N, D_OUT), jnp.bfloat16),
            pltpu.VMEM((ROW_BLOCK, D_OUT), jnp.float32),
        ],
        compiler_params=pltpu.CompilerParams(
            dimension_semantics=("arbitrary",),
        ),
    )(inputs, weight.T, adj_norm)
    return out_t.T
